# baseline (speedup 1.0000x reference)
import functools

import jax
import jax.numpy as jnp
from jax import lax
from jax.experimental import pallas as pl
from jax.experimental.pallas import tpu as pltpu
from jax.experimental.pallas import tpu_sc as plsc

f32 = jnp.float32
i32 = jnp.int32

N = 10000
NP = 10240
E = 320000
NC = 2
NS = 16
NWORK = NC * NS
CHUNK = 128
MROWS = 2560
EPAD = MROWS * CHUNK
RPW = MROWS // NWORK
ROWS_PER_TILE = NP // NS

F1C = 32
F1ROWS = EPAD // F1C
F1RPW = F1ROWS // NWORK
F2C = 64
F2ROWS = EPAD // F2C
F2RPW = F2ROWS // NWORK

_MESH = plsc.VectorSubcoreMesh(core_axis_name="c", subcore_axis_name="s",
                               num_cores=NC, num_subcores=NS)
_CP = pltpu.CompilerParams(needs_layout_passes=False)


def _ids():
    core = lax.axis_index("c")
    sid = lax.axis_index("s")
    wid = sid * NC + core
    return core, sid, wid


def _zero_vmem(ref, rows, width):
    z = jnp.zeros((16,), f32)

    def body(j, _):
        for k in range(width // 16):
            ref[j, pl.ds(k * 16, 16)] = z
        return 0

    lax.fori_loop(0, rows, body, 0, unroll=4)


def _zero_acc(zsrc, rows, acc_sh, sid):
    for t in range(ROWS_PER_TILE // rows):
        pltpu.sync_copy(
            zsrc, acc_sh.at[pl.ds(sid * ROWS_PER_TILE + t * rows, rows)])


F1REAL = E // F1C


@functools.partial(
    pl.kernel,
    out_type=(
        jax.ShapeDtypeStruct((NC, NP, 128), f32),
        jax.ShapeDtypeStruct((NC, NP), f32),
        jax.ShapeDtypeStruct((NC, NP), f32),
    ),
    mesh=_MESH,
    compiler_params=_CP,
    scratch_types=[
        pltpu.VMEM((1, F1C), i32),
        pltpu.VMEM((1, F1C), i32),
        pltpu.VMEM((1, F1C), i32),
        pltpu.VMEM((1, F1C), i32),
        pltpu.VMEM((1, F1C), f32),
        pltpu.VMEM((1, F1C), f32),
        pltpu.VMEM((1, F1C), f32),
        pltpu.VMEM((1, F1C + 16), f32),
        pltpu.VMEM((F1C, 256), f32),
        pltpu.VMEM((F1C, 256), f32),
        pltpu.VMEM((F1C, 128), f32),
        pltpu.VMEM((NP,), f32),
        pltpu.VMEM((NP,), f32),
        pltpu.VMEM_SHARED((NP, 128), f32),
        pltpu.VMEM_SHARED((NP,), f32),
        pltpu.VMEM_SHARED((NP,), f32),
        pltpu.SemaphoreType.DMA,
        pltpu.SemaphoreType.DMA,
        pltpu.SemaphoreType.DMA,
    ],
)
def _f1_kernel(row_hbm, col_hbm, ew_hbm, t1_hbm, s1_hbm, s1m_hbm,
               acc_out, deg_out, cnt_out,
               rowb0, colb0, rowb1, colb1, ewb0, ewb1, onesb, qb, gb0, gb1,
               msgb, s1l, s1ml, acc_sh, deg_sh, cnt_sh, gsem0, gsem1, isem):
    core, sid, wid = _ids()
    pltpu.sync_copy(s1_hbm, s1l)
    pltpu.sync_copy(s1m_hbm, s1ml)
    one = jnp.ones((16,), f32)
    for g in range(F1C // 16):
        onesb[0, pl.ds(g * 16, 16)] = one
    _zero_vmem(msgb, F1C, 128)
    _zero_acc(msgb, F1C, acc_sh, sid)
    for t in range(ROWS_PER_TILE // 128):
        zsl = pl.ds(sid * ROWS_PER_TILE + t * 128, 128)
        pltpu.sync_copy(msgb.at[0], deg_sh.at[zsl])
        pltpu.sync_copy(msgb.at[0], cnt_sh.at[zsl])
    plsc.subcore_barrier()

    sets = ((rowb0, colb0, ewb0, gb0, gsem0), (rowb1, colb1, ewb1, gb1, gsem1))

    def _prefetch(c, s):
        rowb, colb, ewb, gb, gsem = s
        blk = wid * F1RPW + c
        pltpu.async_copy(row_hbm.at[pl.ds(blk, 1)], rowb, isem)
        pltpu.async_copy(col_hbm.at[pl.ds(blk, 1)], colb, isem)
        pltpu.async_copy(ew_hbm.at[pl.ds(blk, 1)], ewb, isem)
        pltpu.make_async_copy(row_hbm.at[pl.ds(blk, 1)], rowb, isem).wait()
        pltpu.make_async_copy(col_hbm.at[pl.ds(blk, 1)], colb, isem).wait()
        pltpu.make_async_copy(ew_hbm.at[pl.ds(blk, 1)], ewb, isem).wait()
        pltpu.async_copy(t1_hbm.at[rowb.at[0]], gb, gsem)

    def _process(c, s):
        rowb, colb, ewb, gb, gsem = s
        blk = wid * F1RPW + c
        for g in range(F1C // 16):
            sl = pl.ds(g * 16, 16)
            sr = plsc.load_gather(s1l, [rowb[0, sl]])
            sc = plsc.load_gather(s1ml, [colb[0, sl]])
            qb[0, sl] = 1.0 / (1.0 + jnp.exp(sc - sr))
        pltpu.make_async_copy(t1_hbm.at[rowb.at[0]], gb, gsem).wait()

        def mbody(j, _):
            q = jnp.full((16,), qb[0, pl.ds(j, 16)][0], f32)
            for k in range(8):
                sl = pl.ds(k * 16, 16)
                msgb[j, sl] = q * gb[j, sl] + gb[j, pl.ds(128 + k * 16, 16)]
            return 0

        lax.fori_loop(0, F1C, mbody, 0, unroll=2)
        pltpu.sync_copy(msgb, acc_sh.at[colb.at[0]], add=True)

        @pl.when(blk < F1REAL)
        def _():
            pltpu.sync_copy(ewb.at[0], deg_sh.at[colb.at[0]], add=True)
            pltpu.sync_copy(onesb.at[0], cnt_sh.at[colb.at[0]], add=True)

    _prefetch(0, sets[0])

    def body(k, _):
        c = 2 * k
        _prefetch(c + 1, sets[1])
        _process(c, sets[0])

        @pl.when(k < F1RPW // 2 - 1)
        def _():
            _prefetch(c + 2, sets[0])

        _process(c + 1, sets[1])
        return 0

    lax.fori_loop(0, F1RPW // 2, body, 0)
    plsc.subcore_barrier()
    sl = pl.ds(sid * ROWS_PER_TILE, ROWS_PER_TILE)
    pltpu.sync_copy(acc_sh.at[sl], acc_out.at[core, sl])
    pltpu.sync_copy(deg_sh.at[sl], deg_out.at[core, sl])
    pltpu.sync_copy(cnt_sh.at[sl], cnt_out.at[core, sl])


@functools.partial(
    pl.kernel,
    out_type=jax.ShapeDtypeStruct((NC, NP, 128), f32),
    mesh=_MESH,
    compiler_params=_CP,
    scratch_types=[
        pltpu.VMEM((1, F2C), i32),
        pltpu.VMEM((1, F2C), i32),
        pltpu.VMEM((1, F2C), i32),
        pltpu.VMEM((1, F2C), i32),
        pltpu.VMEM((1, F2C + 16), f32),
        pltpu.VMEM((F2C, 128), f32),
        pltpu.VMEM((F2C, 128), f32),
        pltpu.VMEM((F2C, 128), f32),
        pltpu.VMEM((NP,), f32),
        pltpu.VMEM((NP,), f32),
        pltpu.VMEM_SHARED((NP, 128), f32),
        pltpu.SemaphoreType.DMA,
        pltpu.SemaphoreType.DMA,
        pltpu.SemaphoreType.DMA,
    ],
)
def _f2_kernel(row_hbm, col_hbm, t2_hbm, s2_hbm, s2m_hbm,
               acc_out,
               rowb0, colb0, rowb1, colb1, qb, gb0, gb1, msgb, s2l, s2ml,
               acc_sh, gsem0, gsem1, isem):
    core, sid, wid = _ids()
    pltpu.sync_copy(s2_hbm, s2l)
    pltpu.sync_copy(s2m_hbm, s2ml)
    _zero_vmem(msgb, F2C, 128)
    _zero_acc(msgb, F2C, acc_sh, sid)
    plsc.subcore_barrier()

    sets = ((rowb0, colb0, gb0, gsem0), (rowb1, colb1, gb1, gsem1))

    def _prefetch(c, s):
        rowb, colb, gb, gsem = s
        blk = wid * F2RPW + c
        pltpu.async_copy(row_hbm.at[pl.ds(blk, 1)], rowb, isem)
        pltpu.async_copy(col_hbm.at[pl.ds(blk, 1)], colb, isem)
        pltpu.make_async_copy(row_hbm.at[pl.ds(blk, 1)], rowb, isem).wait()
        pltpu.make_async_copy(col_hbm.at[pl.ds(blk, 1)], colb, isem).wait()
        pltpu.async_copy(t2_hbm.at[rowb.at[0]], gb, gsem)

    def _process(s):
        rowb, colb, gb, gsem = s
        for g in range(F2C // 16):
            sl = pl.ds(g * 16, 16)
            sr = plsc.load_gather(s2l, [rowb[0, sl]])
            sc = plsc.load_gather(s2ml, [colb[0, sl]])
            qb[0, sl] = 1.0 / (1.0 + jnp.exp(sc - sr))
        pltpu.make_async_copy(t2_hbm.at[rowb.at[0]], gb, gsem).wait()

        def mbody(j, _):
            q = jnp.full((16,), qb[0, pl.ds(j, 16)][0], f32)
            for k in range(4):
                sl = pl.ds(k * 16, 16)
                msgb[j, sl] = q * gb[j, sl] + gb[j, pl.ds(64 + k * 16, 16)]
            return 0

        lax.fori_loop(0, F2C, mbody, 0, unroll=2)
        pltpu.sync_copy(msgb, acc_sh.at[colb.at[0]], add=True)

    _prefetch(0, sets[0])

    def body(k, _):
        c = 2 * k
        _prefetch(c + 1, sets[1])
        _process(sets[0])

        @pl.when(k < F2RPW // 2 - 1)
        def _():
            _prefetch(c + 2, sets[0])

        _process(sets[1])
        return 0

    lax.fori_loop(0, F2RPW // 2, body, 0)
    plsc.subcore_barrier()
    sl = pl.ds(sid * ROWS_PER_TILE, ROWS_PER_TILE)
    pltpu.sync_copy(acc_sh.at[sl], acc_out.at[core, sl])


_GCN_SCRATCH = [
    pltpu.VMEM((1, CHUNK), i32),
    pltpu.VMEM((1, CHUNK), i32),
    pltpu.VMEM((1, CHUNK), i32),
    pltpu.VMEM((1, CHUNK), i32),
    pltpu.VMEM((1, CHUNK + 16), f32),
    pltpu.VMEM((1, CHUNK + 16), f32),
    pltpu.VMEM((CHUNK, 128), f32),
    pltpu.VMEM((CHUNK, 128), f32),
    pltpu.VMEM_SHARED((NP, 128), f32),
    pltpu.SemaphoreType.DMA,
    pltpu.SemaphoreType.DMA,
    pltpu.SemaphoreType.DMA,
]


def _gcn_scale_scatter(s, acc_sh, tab_hbm):
    rowb, colb, nb, gb, gsem = s
    pltpu.make_async_copy(tab_hbm.at[rowb.at[0]], gb, gsem).wait()

    def mbody(j, _):
        nv = jnp.full((16,), nb[0, pl.ds(j, 16)][0], f32)
        for k in range(8):
            sl = pl.ds(k * 16, 16)
            gb[j, sl] = nv * gb[j, sl]
        return 0

    lax.fori_loop(0, CHUNK, mbody, 0, unroll=2)
    pltpu.sync_copy(gb, acc_sh.at[colb.at[0]], add=True)


@functools.partial(
    pl.kernel,
    out_type=(
        jax.ShapeDtypeStruct((NC, NP, 128), f32),
        jax.ShapeDtypeStruct((MROWS, CHUNK), f32),
    ),
    mesh=_MESH,
    compiler_params=_CP,
    scratch_types=_GCN_SCRATCH + [
        pltpu.VMEM((1, CHUNK), f32),
        pltpu.VMEM((1, CHUNK), f32),
        pltpu.VMEM((NP,), f32),
    ],
)
def _gcn_norm_kernel(row_hbm, col_hbm, ew_hbm, dinv_hbm, tab_hbm,
                     acc_out, norm_out,
                     rowb0, colb0, rowb1, colb1, nb0, nb1, gb0, gb1,
                     acc_sh, gsem0, gsem1, isem, ewb0, ewb1, dinvl):
    core, sid, wid = _ids()
    pltpu.sync_copy(dinv_hbm, dinvl)
    _zero_vmem(gb0, CHUNK, 128)
    _zero_acc(gb0, CHUNK, acc_sh, sid)
    plsc.subcore_barrier()

    sets = ((rowb0, colb0, nb0, gb0, gsem0, ewb0),
            (rowb1, colb1, nb1, gb1, gsem1, ewb1))

    def _prefetch(c, s):
        rowb, colb, nb, gb, gsem, ewb = s
        blk = wid * RPW + c
        pltpu.async_copy(row_hbm.at[pl.ds(blk, 1)], rowb, isem)
        pltpu.async_copy(col_hbm.at[pl.ds(blk, 1)], colb, isem)
        pltpu.async_copy(ew_hbm.at[pl.ds(blk, 1)], ewb, isem)
        pltpu.make_async_copy(row_hbm.at[pl.ds(blk, 1)], rowb, isem).wait()
        pltpu.make_async_copy(col_hbm.at[pl.ds(blk, 1)], colb, isem).wait()
        pltpu.make_async_copy(ew_hbm.at[pl.ds(blk, 1)], ewb, isem).wait()
        pltpu.async_copy(tab_hbm.at[rowb.at[0]], gb, gsem)

    def _process(c, s):
        rowb, colb, nb, gb, gsem, ewb = s
        blk = wid * RPW + c
        for g in range(CHUNK // 16):
            sl = pl.ds(g * 16, 16)
            dr = plsc.load_gather(dinvl, [rowb[0, sl]])
            dc = plsc.load_gather(dinvl, [colb[0, sl]])
            nb[0, sl] = dr * ewb[0, sl] * dc
        _gcn_scale_scatter((rowb, colb, nb, gb, gsem), acc_sh, tab_hbm)
        pltpu.sync_copy(nb.at[:, pl.ds(0, CHUNK)], norm_out.at[pl.ds(blk, 1)])

    _prefetch(0, sets[0])

    def body(k, _):
        c = 2 * k
        _prefetch(c + 1, sets[1])
        _process(c, sets[0])

        @pl.when(k < RPW // 2 - 1)
        def _():
            _prefetch(c + 2, sets[0])

        _process(c + 1, sets[1])
        return 0

    lax.fori_loop(0, RPW // 2, body, 0)
    plsc.subcore_barrier()
    sl = pl.ds(sid * ROWS_PER_TILE, ROWS_PER_TILE)
    pltpu.sync_copy(acc_sh.at[sl], acc_out.at[core, sl])


@functools.partial(
    pl.kernel,
    out_type=jax.ShapeDtypeStruct((NC, NP, 128), f32),
    mesh=_MESH,
    compiler_params=_CP,
    scratch_types=_GCN_SCRATCH,
)
def _gcn_kernel(row_hbm, col_hbm, norm_hbm, tab_hbm,
                acc_out,
                rowb0, colb0, rowb1, colb1, nb0, nb1, gb0, gb1,
                acc_sh, gsem0, gsem1, isem):
    core, sid, wid = _ids()
    _zero_vmem(gb0, CHUNK, 128)
    _zero_acc(gb0, CHUNK, acc_sh, sid)
    plsc.subcore_barrier()

    sets = ((rowb0, colb0, nb0, gb0, gsem0), (rowb1, colb1, nb1, gb1, gsem1))

    def _prefetch(c, s):
        rowb, colb, nb, gb, gsem = s
        blk = wid * RPW + c
        pltpu.async_copy(row_hbm.at[pl.ds(blk, 1)], rowb, isem)
        pltpu.async_copy(col_hbm.at[pl.ds(blk, 1)], colb, isem)
        pltpu.async_copy(norm_hbm.at[pl.ds(blk, 1)],
                         nb.at[:, pl.ds(0, CHUNK)], isem)
        pltpu.make_async_copy(row_hbm.at[pl.ds(blk, 1)], rowb, isem).wait()
        pltpu.make_async_copy(col_hbm.at[pl.ds(blk, 1)], colb, isem).wait()
        pltpu.make_async_copy(norm_hbm.at[pl.ds(blk, 1)],
                              nb.at[:, pl.ds(0, CHUNK)], isem).wait()
        pltpu.async_copy(tab_hbm.at[rowb.at[0]], gb, gsem)

    _prefetch(0, sets[0])

    def body(k, _):
        c = 2 * k
        _prefetch(c + 1, sets[1])
        _gcn_scale_scatter(sets[0], acc_sh, tab_hbm)

        @pl.when(k < RPW // 2 - 1)
        def _():
            _prefetch(c + 2, sets[0])

        _gcn_scale_scatter(sets[1], acc_sh, tab_hbm)
        return 0

    lax.fori_loop(0, RPW // 2, body, 0)
    plsc.subcore_barrier()
    sl = pl.ds(sid * ROWS_PER_TILE, ROWS_PER_TILE)
    pltpu.sync_copy(acc_sh.at[sl], acc_out.at[core, sl])


@functools.partial(
    pl.kernel,
    out_type=jax.ShapeDtypeStruct((MROWS, CHUNK), f32),
    mesh=_MESH,
    compiler_params=_CP,
    scratch_types=[
        pltpu.VMEM((1, CHUNK), i32),
        pltpu.VMEM((1, CHUNK), i32),
        pltpu.VMEM((1, CHUNK), i32),
        pltpu.VMEM((1, CHUNK), i32),
        pltpu.VMEM((1, CHUNK), f32),
        pltpu.VMEM((1, CHUNK), f32),
        pltpu.VMEM((1, CHUNK), f32),
        pltpu.VMEM((CHUNK, 128), f32),
        pltpu.VMEM((CHUNK, 128), f32),
        pltpu.VMEM((CHUNK, 128), f32),
        pltpu.VMEM((CHUNK, 128), f32),
        pltpu.VMEM((CHUNK, 16), f32),
        pltpu.VMEM((128,), f32),
        pltpu.SemaphoreType.DMA,
        pltpu.SemaphoreType.DMA,
        pltpu.SemaphoreType.DMA,
    ],
)
def _pet_kernel(row_hbm, col_hbm, ewt_hbm, h_hbm, w_hbm,
                pet_out,
                rowb0, colb0, rowb1, colb1, ewtb0, ewtb1, petb,
                hrb0, hcb0, hrb1, hcb1, sb, wb, gsem0, gsem1, isem):
    core, sid, wid = _ids()
    pltpu.sync_copy(w_hbm, wb)
    wv = [wb[pl.ds(k * 16, 16)] for k in range(8)]
    lanes = lax.iota(i32, 16)

    sets = ((rowb0, colb0, ewtb0, hrb0, hcb0, gsem0),
            (rowb1, colb1, ewtb1, hrb1, hcb1, gsem1))

    def _prefetch(c, s):
        rowb, colb, ewtb, hrb, hcb, gsem = s
        blk = wid * RPW + c
        pltpu.async_copy(row_hbm.at[pl.ds(blk, 1)], rowb, isem)
        pltpu.async_copy(col_hbm.at[pl.ds(blk, 1)], colb, isem)
        pltpu.async_copy(ewt_hbm.at[pl.ds(blk, 1)], ewtb, isem)
        pltpu.make_async_copy(row_hbm.at[pl.ds(blk, 1)], rowb, isem).wait()
        pltpu.make_async_copy(col_hbm.at[pl.ds(blk, 1)], colb, isem).wait()
        pltpu.make_async_copy(ewt_hbm.at[pl.ds(blk, 1)], ewtb, isem).wait()
        pltpu.async_copy(h_hbm.at[rowb.at[0]], hrb, gsem)
        pltpu.async_copy(h_hbm.at[colb.at[0]], hcb, gsem)

    def _process(c, s):
        rowb, colb, ewtb, hrb, hcb, gsem = s
        blk = wid * RPW + c
        pltpu.make_async_copy(h_hbm.at[rowb.at[0]], hrb, gsem).wait()
        pltpu.make_async_copy(h_hbm.at[colb.at[0]], hcb, gsem).wait()

        def pbody(j, _):
            s16 = jnp.zeros((16,), f32)
            for k in range(8):
                sl = pl.ds(k * 16, 16)
                s16 = s16 + jnp.abs(hrb[j, sl] - hcb[j, sl]) * wv[k]
            sb[j, :] = s16
            return 0

        lax.fori_loop(0, CHUNK, pbody, 0, unroll=2)
        for g in range(CHUNK // 16):
            eidx = g * 16 + lanes
            tot = jnp.zeros((16,), f32)
            for k in range(16):
                tot = tot + plsc.load_gather(
                    sb, [eidx, jnp.full((16,), k, i32)])
            sl = pl.ds(g * 16, 16)
            petb[0, sl] = tot + ewtb[0, sl]
        pltpu.sync_copy(petb, pet_out.at[pl.ds(blk, 1)])

    _prefetch(0, sets[0])

    def body(k, _):
        c = 2 * k
        _prefetch(c + 1, sets[1])
        _process(c, sets[0])

        @pl.when(k < RPW // 2 - 1)
        def _():
            _prefetch(c + 2, sets[0])

        _process(c + 1, sets[1])
        return 0

    lax.fori_loop(0, RPW // 2, body, 0)


def _tc(body, out_shapes, *ins):
    return pl.pallas_call(body, out_shape=out_shapes)(*ins)


def _tca_body(x_ref, du_ref, w_ref, pv_ref, ew_ref,
              s1_ref, s1m_ref, t1_ref, self1_ref, ewt_ref):
    x = x_ref[...]
    s1 = jnp.dot(x, du_ref[...], preferred_element_type=f32)
    s1_ref[...] = s1
    s1m_ref[...] = s1 - pv_ref[0]
    xw = jnp.dot(x, w_ref[...], preferred_element_type=f32)
    m0 = xw[:, :128]
    m1 = xw[:, 128:]
    t1_ref[...] = jnp.concatenate([m0 - m1, m1], axis=1)
    self1_ref[...] = pv_ref[1] * m0 + pv_ref[2] * m1
    ewt_ref[...] = ew_ref[...] * pv_ref[3] + pv_ref[4]


def _tcb_body(acc_ref, self1_ref, degE_ref, cntE_ref, b_ref, du_ref, w_ref,
              pv_ref,
              dinv_ref, cnt_ref, s2_ref, s2m_ref, t2_ref, self2_ref):
    deg = degE_ref[0] + degE_ref[1] + 1.0
    dinv_ref[...] = jnp.where(deg > 0, lax.rsqrt(deg), 0.0)
    cnt0 = cntE_ref[0] + cntE_ref[1] + 1.0
    cnt_ref[...] = cnt0
    cnt = jnp.maximum(cnt0, 1.0)
    h1 = (acc_ref[0] + acc_ref[1] + self1_ref[...]) / cnt[:, None]
    h1 = jnp.maximum(h1 + b_ref[...][None, :], 0.0)
    s2 = jnp.dot(h1, du_ref[...], preferred_element_type=f32)
    s2_ref[...] = s2
    s2m_ref[...] = s2 - pv_ref[0]
    xw = jnp.dot(h1, w_ref[...], preferred_element_type=f32)
    m0 = xw[:, :64]
    m1 = xw[:, 64:]
    t2_ref[...] = jnp.concatenate([m0 - m1, m1], axis=1)
    self2_ref[...] = pv_ref[1] * m0 + pv_ref[2] * m1


def _tcc_body(acc_ref, self2_ref, cnt_ref, b2_ref, wl_ref, bl_ref, we_ref,
              be_ref, dinv_ref,
              h_ref, hw_ref, selfb_ref):
    cnt = jnp.maximum(cnt_ref[...], 1.0)
    h2 = (acc_ref[0, :, :64] + acc_ref[1, :, :64]
          + self2_ref[...]) / cnt[:, None]
    h2 = jnp.maximum(h2 + b2_ref[...][None, :], 0.0)
    h = jnp.dot(h2, wl_ref[...], preferred_element_type=f32) \
        + bl_ref[...][None, :]
    h_ref[...] = h
    hw = jnp.dot(h, we_ref[...], preferred_element_type=f32)
    hw_ref[...] = hw
    d2 = dinv_ref[...] * dinv_ref[...]
    selfb_ref[...] = d2[:, None] * hw + be_ref[...][None, :]


def _tcg_body(acc_ref, selfb_ref, w_ref, b_ref, dinv_ref,
              hw_ref, selfb2_ref):
    e = jnp.maximum(acc_ref[0] + acc_ref[1] + selfb_ref[...], 0.0)
    hw = jnp.dot(e, w_ref[...], preferred_element_type=f32)
    hw_ref[...] = hw
    d2 = dinv_ref[...] * dinv_ref[...]
    selfb2_ref[...] = d2[:, None] * hw + b_ref[...][None, :]


def _tce_body(acc_ref, selfb_ref, w_ref, b_ref, dinv_ref,
              mu_ref, lv_ref, hw_ref, selfb2_ref):
    e = jnp.maximum(acc_ref[0] + acc_ref[1] + selfb_ref[...], 0.0)
    mu = e[:, :64]
    mu_ref[...] = mu
    lv_ref[...] = e[:, 64:]
    hw = jnp.dot(mu, w_ref[...], preferred_element_type=f32)
    hw_ref[...] = hw
    d2 = dinv_ref[...] * dinv_ref[...]
    selfb2_ref[...] = d2[:, None] * hw + b_ref[...][None, :]


def _tcf_body(acc_ref, selfb_ref, recon_ref):
    recon_ref[...] = jnp.tanh(acc_ref[0] + acc_ref[1] + selfb_ref[...])


def kernel(x, edge_index, edge_weight, params):
    n, df = x.shape
    e = edge_index.shape[1]

    pad_e = EPAD - e
    row = jnp.concatenate(
        [edge_index[0], jnp.full((pad_e,), n, i32)]).reshape(MROWS, CHUNK)
    col = jnp.concatenate(
        [edge_index[1], jnp.full((pad_e,), n, i32)]).reshape(MROWS, CHUNK)
    ew2 = jnp.concatenate(
        [edge_weight[:, 0], jnp.zeros((pad_e,), f32)]).reshape(MROWS, CHUNK)
    xp = jnp.pad(x, ((0, NP - n), (0, 0)))

    def _mat(p, key):
        w = p['w_mu'] + jnp.exp(0.5 * p['w_logvar']) * jax.random.normal(
            key, p['w_mu'].shape, dtype=f32)
        b = p['b_mu'] + jnp.exp(0.5 * p['b_logvar']) * jax.random.normal(
            jax.random.fold_in(key, 1), p['b_mu'].shape, dtype=f32)
        return w, b

    kk = jax.random.key(42)
    we1, be1 = _mat(params['enc1'], jax.random.fold_in(kk, 0))
    we2, be2 = _mat(params['enc2'], jax.random.fold_in(kk, 1))
    wd1, bd1 = _mat(params['dec1'], jax.random.fold_in(kk, 2))
    wd2, bd2 = _mat(params['dec2'], jax.random.fold_in(kk, 3))

    f1, f2 = params['feast1'], params['feast2']
    du1 = f1['u'][:, 0] - f1['u'][:, 1]
    q1 = jax.nn.softmax(f1['c'])
    du2 = f2['u'][:, 0] - f2['u'][:, 1]
    q2 = jax.nn.softmax(f2['c'])
    etpw = params['etp']['W']
    pva = jnp.stack([f1['c'][0] - f1['c'][1], q1[0], q1[1],
                     etpw[df, 0], params['etp']['b'][0]])
    pvb = jnp.stack([f2['c'][0] - f2['c'][1], q2[0], q2[1]])
    wvec = etpw[:df, 0]

    s1, s1m, t1, self1, ewt = _tc(
        _tca_body,
        (jax.ShapeDtypeStruct((NP,), f32),
         jax.ShapeDtypeStruct((NP,), f32),
         jax.ShapeDtypeStruct((NP, 256), f32),
         jax.ShapeDtypeStruct((NP, 128), f32),
         jax.ShapeDtypeStruct((MROWS, CHUNK), f32)),
        xp, du1, f1['W'], pva, ew2)

    acc1, degE, cntE = _f1_kernel(
        row.reshape(F1ROWS, F1C), col.reshape(F1ROWS, F1C),
        ew2.reshape(F1ROWS, F1C), t1, s1, s1m)

    dinv, cnt, s2, s2m, t2, self2 = _tc(
        _tcb_body,
        (jax.ShapeDtypeStruct((NP,), f32),
         jax.ShapeDtypeStruct((NP,), f32),
         jax.ShapeDtypeStruct((NP,), f32),
         jax.ShapeDtypeStruct((NP,), f32),
         jax.ShapeDtypeStruct((NP, 128), f32),
         jax.ShapeDtypeStruct((NP, 64), f32)),
        acc1, self1, degE, cntE, f1['b'], du2, f2['W'], pvb)

    acc2 = _f2_kernel(row.reshape(F2ROWS, F2C), col.reshape(F2ROWS, F2C),
                      t2, s2, s2m)

    h, hw1, selfb1 = _tc(
        _tcc_body,
        (jax.ShapeDtypeStruct((NP, 128), f32),
         jax.ShapeDtypeStruct((NP, 128), f32),
         jax.ShapeDtypeStruct((NP, 128), f32)),
        acc2, self2, cnt, f2['b'], params['linear']['W'],
        params['linear']['b'], we1, be1, dinv)

    accg1, norm = _gcn_norm_kernel(row, col, ew2, dinv, hw1)
    pet2 = _pet_kernel(row, col, ewt, h, wvec)

    hw2, selfb2 = _tc(
        _tcg_body,
        (jax.ShapeDtypeStruct((NP, 128), f32),
         jax.ShapeDtypeStruct((NP, 128), f32)),
        accg1, selfb1, we2, be2, dinv)

    accg2 = _gcn_kernel(row, col, norm, hw2)

    mu, logvar, hw3, selfb3 = _tc(
        _tce_body,
        (jax.ShapeDtypeStruct((NP, 64), f32),
         jax.ShapeDtypeStruct((NP, 64), f32),
         jax.ShapeDtypeStruct((NP, 128), f32),
         jax.ShapeDtypeStruct((NP, 128), f32)),
        accg2, selfb2, wd1, bd1, dinv)

    accg3 = _gcn_kernel(row, col, norm, hw3)

    hw4, selfb4 = _tc(
        _tcg_body,
        (jax.ShapeDtypeStruct((NP, 128), f32),
         jax.ShapeDtypeStruct((NP, 128), f32)),
        accg3, selfb3, wd2, bd2, dinv)

    accg4 = _gcn_kernel(row, col, norm, hw4)

    recon = _tc(
        _tcf_body,
        jax.ShapeDtypeStruct((NP, 128), f32),
        accg4, selfb4)

    pet = pet2.reshape(-1)[:e, None]
    return recon[:n], mu[:n], logvar[:n], pet


# speedup vs baseline: 1.0677x; 1.0677x over previous
import functools

import jax
import jax.numpy as jnp
from jax import lax
from jax.experimental import pallas as pl
from jax.experimental.pallas import tpu as pltpu
from jax.experimental.pallas import tpu_sc as plsc

f32 = jnp.float32
i32 = jnp.int32

N = 10000
NP = 10240
E = 320000
NC = 2
NS = 16
NWORK = NC * NS
CHUNK = 128
MROWS = 2560
EPAD = MROWS * CHUNK
RPW = MROWS // NWORK
ROWS_PER_TILE = NP // NS

F1C = 32
F1ROWS = EPAD // F1C
F1RPW = F1ROWS // NWORK
F2C = 64
F2ROWS = EPAD // F2C
F2RPW = F2ROWS // NWORK

_MESH = plsc.VectorSubcoreMesh(core_axis_name="c", subcore_axis_name="s",
                               num_cores=NC, num_subcores=NS)
_CP = pltpu.CompilerParams(needs_layout_passes=False)


def _ids():
    core = lax.axis_index("c")
    sid = lax.axis_index("s")
    wid = sid * NC + core
    return core, sid, wid


def _zero_vmem(ref, rows, width):
    z = jnp.zeros((16,), f32)

    def body(j, _):
        for k in range(width // 16):
            ref[j, pl.ds(k * 16, 16)] = z
        return 0

    lax.fori_loop(0, rows, body, 0, unroll=4)


def _zero_acc(zsrc, rows, acc_sh, sid):
    for t in range(ROWS_PER_TILE // rows):
        pltpu.sync_copy(
            zsrc, acc_sh.at[pl.ds(sid * ROWS_PER_TILE + t * rows, rows)])


@functools.partial(
    pl.kernel,
    out_type=(
        jax.ShapeDtypeStruct((NC, NP), f32),
        jax.ShapeDtypeStruct((NC, NP), f32),
    ),
    mesh=_MESH,
    compiler_params=_CP,
    scratch_types=[
        pltpu.VMEM((1, CHUNK), i32),
        pltpu.VMEM((1, CHUNK), i32),
        pltpu.VMEM((2, CHUNK), f32),
        pltpu.VMEM((2, CHUNK), f32),
        pltpu.VMEM((ROWS_PER_TILE,), f32),
        pltpu.VMEM_SHARED((NP,), f32),
        pltpu.VMEM_SHARED((NP,), f32),
        pltpu.SemaphoreType.DMA,
        pltpu.SemaphoreType.DMA,
    ],
)
def _deg_kernel(col_hbm, wv_hbm, deg_out, cnt_out,
                colb0, colb1, wvb0, wvb1, zb, deg_sh, cnt_sh, sem0, sem1):
    core, sid, wid = _ids()
    z = jnp.zeros((16,), f32)

    def zbody(j, _):
        zb[pl.ds(j * 16, 16)] = z
        return 0

    lax.fori_loop(0, ROWS_PER_TILE // 16, zbody, 0, unroll=4)
    pltpu.sync_copy(zb, deg_sh.at[pl.ds(sid * ROWS_PER_TILE, ROWS_PER_TILE)])
    pltpu.sync_copy(zb, cnt_sh.at[pl.ds(sid * ROWS_PER_TILE, ROWS_PER_TILE)])
    plsc.subcore_barrier()

    sets = ((colb0, wvb0, sem0), (colb1, wvb1, sem1))

    def _prefetch(c, s):
        colb, wvb, sem = s
        blk = wid * RPW + c
        pltpu.async_copy(col_hbm.at[pl.ds(blk, 1)], colb, sem)
        pltpu.async_copy(wv_hbm.at[pl.ds(2 * blk, 2)], wvb, sem)

    def _drain(s):
        colb, wvb, sem = s
        pltpu.make_async_copy(col_hbm.at[pl.ds(0, 1)], colb, sem).wait()
        pltpu.make_async_copy(wv_hbm.at[pl.ds(0, 2)], wvb, sem).wait()

    def _process(s):
        colb, wvb, _ = s
        pltpu.sync_copy(wvb.at[0], deg_sh.at[colb.at[0]], add=True)
        pltpu.sync_copy(wvb.at[1], cnt_sh.at[colb.at[0]], add=True)

    _prefetch(0, sets[0])

    def body(k, _):
        c = 2 * k
        _prefetch(c + 1, sets[1])
        _drain(sets[0])
        _process(sets[0])

        @pl.when(k < RPW // 2 - 1)
        def _():
            _prefetch(c + 2, sets[0])

        _drain(sets[1])
        _process(sets[1])
        return 0

    lax.fori_loop(0, RPW // 2, body, 0)
    plsc.subcore_barrier()
    sl = pl.ds(sid * ROWS_PER_TILE, ROWS_PER_TILE)
    pltpu.sync_copy(deg_sh.at[sl], deg_out.at[core, sl])
    pltpu.sync_copy(cnt_sh.at[sl], cnt_out.at[core, sl])


F1REAL = E // F1C


@functools.partial(
    pl.kernel,
    out_type=jax.ShapeDtypeStruct((NC, NP, 128), f32),
    mesh=_MESH,
    compiler_params=_CP,
    scratch_types=[
        pltpu.VMEM((1, F1C), i32),
        pltpu.VMEM((1, F1C), i32),
        pltpu.VMEM((1, F1C), i32),
        pltpu.VMEM((1, F1C), i32),
        pltpu.VMEM((1, F1C + 16), f32),
        pltpu.VMEM((F1C, 256), f32),
        pltpu.VMEM((F1C, 256), f32),
        pltpu.VMEM((F1C, 128), f32),
        pltpu.VMEM((NP,), f32),
        pltpu.VMEM((NP,), f32),
        pltpu.VMEM_SHARED((NP, 128), f32),
        pltpu.SemaphoreType.DMA,
        pltpu.SemaphoreType.DMA,
        pltpu.SemaphoreType.DMA,
    ],
)
def _f1_kernel(row_hbm, col_hbm, t1_hbm, s1_hbm, s1m_hbm,
               acc_out,
               rowb0, colb0, rowb1, colb1, qb, gb0, gb1,
               msgb, s1l, s1ml, acc_sh, gsem0, gsem1, isem):
    core, sid, wid = _ids()
    pltpu.sync_copy(s1_hbm, s1l)
    pltpu.sync_copy(s1m_hbm, s1ml)
    _zero_vmem(msgb, F1C, 128)
    _zero_acc(msgb, F1C, acc_sh, sid)
    plsc.subcore_barrier()

    sets = ((rowb0, colb0, gb0, gsem0), (rowb1, colb1, gb1, gsem1))

    def _prefetch(c, s):
        rowb, colb, gb, gsem = s
        blk = wid * F1RPW + c
        pltpu.async_copy(row_hbm.at[pl.ds(blk, 1)], rowb, isem)
        pltpu.async_copy(col_hbm.at[pl.ds(blk, 1)], colb, isem)
        pltpu.make_async_copy(row_hbm.at[pl.ds(blk, 1)], rowb, isem).wait()
        pltpu.make_async_copy(col_hbm.at[pl.ds(blk, 1)], colb, isem).wait()
        pltpu.async_copy(t1_hbm.at[rowb.at[0]], gb, gsem)

    def _process(c, s):
        rowb, colb, gb, gsem = s
        for g in range(F1C // 16):
            sl = pl.ds(g * 16, 16)
            sr = plsc.load_gather(s1l, [rowb[0, sl]])
            sc = plsc.load_gather(s1ml, [colb[0, sl]])
            qb[0, sl] = 1.0 / (1.0 + jnp.exp(sc - sr))
        pltpu.make_async_copy(t1_hbm.at[rowb.at[0]], gb, gsem).wait()

        def mbody(j, _):
            q = jnp.full((16,), qb[0, pl.ds(j, 16)][0], f32)
            for k in range(8):
                sl = pl.ds(k * 16, 16)
                msgb[j, sl] = q * gb[j, sl] + gb[j, pl.ds(128 + k * 16, 16)]
            return 0

        lax.fori_loop(0, F1C, mbody, 0, unroll=2)
        pltpu.sync_copy(msgb, acc_sh.at[colb.at[0]], add=True)

    _prefetch(0, sets[0])

    def body(k, _):
        c = 2 * k
        _prefetch(c + 1, sets[1])
        _process(c, sets[0])

        @pl.when(k < F1RPW // 2 - 1)
        def _():
            _prefetch(c + 2, sets[0])

        _process(c + 1, sets[1])
        return 0

    lax.fori_loop(0, F1RPW // 2, body, 0)
    plsc.subcore_barrier()
    sl = pl.ds(sid * ROWS_PER_TILE, ROWS_PER_TILE)
    pltpu.sync_copy(acc_sh.at[sl], acc_out.at[core, sl])


@functools.partial(
    pl.kernel,
    out_type=jax.ShapeDtypeStruct((NC, NP, 128), f32),
    mesh=_MESH,
    compiler_params=_CP,
    scratch_types=[
        pltpu.VMEM((1, F2C), i32),
        pltpu.VMEM((1, F2C), i32),
        pltpu.VMEM((1, F2C), i32),
        pltpu.VMEM((1, F2C), i32),
        pltpu.VMEM((1, F2C + 16), f32),
        pltpu.VMEM((F2C, 128), f32),
        pltpu.VMEM((F2C, 128), f32),
        pltpu.VMEM((F2C, 128), f32),
        pltpu.VMEM((NP,), f32),
        pltpu.VMEM((NP,), f32),
        pltpu.VMEM_SHARED((NP, 128), f32),
        pltpu.SemaphoreType.DMA,
        pltpu.SemaphoreType.DMA,
        pltpu.SemaphoreType.DMA,
    ],
)
def _f2_kernel(row_hbm, col_hbm, t2_hbm, s2_hbm, s2m_hbm,
               acc_out,
               rowb0, colb0, rowb1, colb1, qb, gb0, gb1, msgb, s2l, s2ml,
               acc_sh, gsem0, gsem1, isem):
    core, sid, wid = _ids()
    pltpu.sync_copy(s2_hbm, s2l)
    pltpu.sync_copy(s2m_hbm, s2ml)
    _zero_vmem(msgb, F2C, 128)
    _zero_acc(msgb, F2C, acc_sh, sid)
    plsc.subcore_barrier()

    sets = ((rowb0, colb0, gb0, gsem0), (rowb1, colb1, gb1, gsem1))

    def _prefetch(c, s):
        rowb, colb, gb, gsem = s
        blk = wid * F2RPW + c
        pltpu.async_copy(row_hbm.at[pl.ds(blk, 1)], rowb, isem)
        pltpu.async_copy(col_hbm.at[pl.ds(blk, 1)], colb, isem)
        pltpu.make_async_copy(row_hbm.at[pl.ds(blk, 1)], rowb, isem).wait()
        pltpu.make_async_copy(col_hbm.at[pl.ds(blk, 1)], colb, isem).wait()
        pltpu.async_copy(t2_hbm.at[rowb.at[0]], gb, gsem)

    def _process(s):
        rowb, colb, gb, gsem = s
        for g in range(F2C // 16):
            sl = pl.ds(g * 16, 16)
            sr = plsc.load_gather(s2l, [rowb[0, sl]])
            sc = plsc.load_gather(s2ml, [colb[0, sl]])
            qb[0, sl] = 1.0 / (1.0 + jnp.exp(sc - sr))
        pltpu.make_async_copy(t2_hbm.at[rowb.at[0]], gb, gsem).wait()

        def mbody(j, _):
            q = jnp.full((16,), qb[0, pl.ds(j, 16)][0], f32)
            for k in range(4):
                sl = pl.ds(k * 16, 16)
                msgb[j, sl] = q * gb[j, sl] + gb[j, pl.ds(64 + k * 16, 16)]
            return 0

        lax.fori_loop(0, F2C, mbody, 0, unroll=2)
        pltpu.sync_copy(msgb, acc_sh.at[colb.at[0]], add=True)

    _prefetch(0, sets[0])

    def body(k, _):
        c = 2 * k
        _prefetch(c + 1, sets[1])
        _process(sets[0])

        @pl.when(k < F2RPW // 2 - 1)
        def _():
            _prefetch(c + 2, sets[0])

        _process(sets[1])
        return 0

    lax.fori_loop(0, F2RPW // 2, body, 0)
    plsc.subcore_barrier()
    sl = pl.ds(sid * ROWS_PER_TILE, ROWS_PER_TILE)
    pltpu.sync_copy(acc_sh.at[sl], acc_out.at[core, sl])


_GCN_SCRATCH = [
    pltpu.VMEM((1, CHUNK), i32),
    pltpu.VMEM((1, CHUNK), i32),
    pltpu.VMEM((1, CHUNK), i32),
    pltpu.VMEM((1, CHUNK), i32),
    pltpu.VMEM((1, CHUNK + 16), f32),
    pltpu.VMEM((1, CHUNK + 16), f32),
    pltpu.VMEM((CHUNK, 128), f32),
    pltpu.VMEM((CHUNK, 128), f32),
    pltpu.VMEM_SHARED((NP, 128), f32),
    pltpu.SemaphoreType.DMA,
    pltpu.SemaphoreType.DMA,
    pltpu.SemaphoreType.DMA,
]


def _gcn_scale_scatter(s, acc_sh, tab_hbm):
    rowb, colb, nb, gb, gsem = s
    pltpu.make_async_copy(tab_hbm.at[rowb.at[0]], gb, gsem).wait()

    def mbody(j, _):
        nv = jnp.full((16,), nb[0, pl.ds(j, 16)][0], f32)
        for k in range(8):
            sl = pl.ds(k * 16, 16)
            gb[j, sl] = nv * gb[j, sl]
        return 0

    lax.fori_loop(0, CHUNK, mbody, 0, unroll=2)
    pltpu.sync_copy(gb, acc_sh.at[colb.at[0]], add=True)


@functools.partial(
    pl.kernel,
    out_type=(
        jax.ShapeDtypeStruct((NC, NP, 128), f32),
        jax.ShapeDtypeStruct((MROWS, CHUNK), f32),
    ),
    mesh=_MESH,
    compiler_params=_CP,
    scratch_types=_GCN_SCRATCH + [
        pltpu.VMEM((1, CHUNK), f32),
        pltpu.VMEM((1, CHUNK), f32),
        pltpu.VMEM((NP,), f32),
    ],
)
def _gcn_norm_kernel(row_hbm, col_hbm, ew_hbm, dinv_hbm, tab_hbm,
                     acc_out, norm_out,
                     rowb0, colb0, rowb1, colb1, nb0, nb1, gb0, gb1,
                     acc_sh, gsem0, gsem1, isem, ewb0, ewb1, dinvl):
    core, sid, wid = _ids()
    pltpu.sync_copy(dinv_hbm, dinvl)
    _zero_vmem(gb0, CHUNK, 128)
    _zero_acc(gb0, CHUNK, acc_sh, sid)
    plsc.subcore_barrier()

    sets = ((rowb0, colb0, nb0, gb0, gsem0, ewb0),
            (rowb1, colb1, nb1, gb1, gsem1, ewb1))

    def _prefetch(c, s):
        rowb, colb, nb, gb, gsem, ewb = s
        blk = wid * RPW + c
        pltpu.async_copy(row_hbm.at[pl.ds(blk, 1)], rowb, isem)
        pltpu.async_copy(col_hbm.at[pl.ds(blk, 1)], colb, isem)
        pltpu.async_copy(ew_hbm.at[pl.ds(blk, 1)], ewb, isem)
        pltpu.make_async_copy(row_hbm.at[pl.ds(blk, 1)], rowb, isem).wait()
        pltpu.make_async_copy(col_hbm.at[pl.ds(blk, 1)], colb, isem).wait()
        pltpu.make_async_copy(ew_hbm.at[pl.ds(blk, 1)], ewb, isem).wait()
        pltpu.async_copy(tab_hbm.at[rowb.at[0]], gb, gsem)

    def _process(c, s):
        rowb, colb, nb, gb, gsem, ewb = s
        blk = wid * RPW + c
        for g in range(CHUNK // 16):
            sl = pl.ds(g * 16, 16)
            dr = plsc.load_gather(dinvl, [rowb[0, sl]])
            dc = plsc.load_gather(dinvl, [colb[0, sl]])
            nb[0, sl] = dr * ewb[0, sl] * dc
        _gcn_scale_scatter((rowb, colb, nb, gb, gsem), acc_sh, tab_hbm)
        pltpu.sync_copy(nb.at[:, pl.ds(0, CHUNK)], norm_out.at[pl.ds(blk, 1)])

    _prefetch(0, sets[0])

    def body(k, _):
        c = 2 * k
        _prefetch(c + 1, sets[1])
        _process(c, sets[0])

        @pl.when(k < RPW // 2 - 1)
        def _():
            _prefetch(c + 2, sets[0])

        _process(c + 1, sets[1])
        return 0

    lax.fori_loop(0, RPW // 2, body, 0)
    plsc.subcore_barrier()
    sl = pl.ds(sid * ROWS_PER_TILE, ROWS_PER_TILE)
    pltpu.sync_copy(acc_sh.at[sl], acc_out.at[core, sl])


@functools.partial(
    pl.kernel,
    out_type=jax.ShapeDtypeStruct((NC, NP, 128), f32),
    mesh=_MESH,
    compiler_params=_CP,
    scratch_types=_GCN_SCRATCH,
)
def _gcn_kernel(row_hbm, col_hbm, norm_hbm, tab_hbm,
                acc_out,
                rowb0, colb0, rowb1, colb1, nb0, nb1, gb0, gb1,
                acc_sh, gsem0, gsem1, isem):
    core, sid, wid = _ids()
    _zero_vmem(gb0, CHUNK, 128)
    _zero_acc(gb0, CHUNK, acc_sh, sid)
    plsc.subcore_barrier()

    sets = ((rowb0, colb0, nb0, gb0, gsem0), (rowb1, colb1, nb1, gb1, gsem1))

    def _prefetch(c, s):
        rowb, colb, nb, gb, gsem = s
        blk = wid * RPW + c
        pltpu.async_copy(row_hbm.at[pl.ds(blk, 1)], rowb, isem)
        pltpu.async_copy(col_hbm.at[pl.ds(blk, 1)], colb, isem)
        pltpu.async_copy(norm_hbm.at[pl.ds(blk, 1)],
                         nb.at[:, pl.ds(0, CHUNK)], isem)
        pltpu.make_async_copy(row_hbm.at[pl.ds(blk, 1)], rowb, isem).wait()
        pltpu.make_async_copy(col_hbm.at[pl.ds(blk, 1)], colb, isem).wait()
        pltpu.make_async_copy(norm_hbm.at[pl.ds(blk, 1)],
                              nb.at[:, pl.ds(0, CHUNK)], isem).wait()
        pltpu.async_copy(tab_hbm.at[rowb.at[0]], gb, gsem)

    _prefetch(0, sets[0])

    def body(k, _):
        c = 2 * k
        _prefetch(c + 1, sets[1])
        _gcn_scale_scatter(sets[0], acc_sh, tab_hbm)

        @pl.when(k < RPW // 2 - 1)
        def _():
            _prefetch(c + 2, sets[0])

        _gcn_scale_scatter(sets[1], acc_sh, tab_hbm)
        return 0

    lax.fori_loop(0, RPW // 2, body, 0)
    plsc.subcore_barrier()
    sl = pl.ds(sid * ROWS_PER_TILE, ROWS_PER_TILE)
    pltpu.sync_copy(acc_sh.at[sl], acc_out.at[core, sl])


@functools.partial(
    pl.kernel,
    out_type=jax.ShapeDtypeStruct((MROWS, CHUNK), f32),
    mesh=_MESH,
    compiler_params=_CP,
    scratch_types=[
        pltpu.VMEM((1, CHUNK), i32),
        pltpu.VMEM((1, CHUNK), i32),
        pltpu.VMEM((1, CHUNK), i32),
        pltpu.VMEM((1, CHUNK), i32),
        pltpu.VMEM((1, CHUNK), f32),
        pltpu.VMEM((1, CHUNK), f32),
        pltpu.VMEM((1, CHUNK), f32),
        pltpu.VMEM((CHUNK, 128), f32),
        pltpu.VMEM((CHUNK, 128), f32),
        pltpu.VMEM((CHUNK, 128), f32),
        pltpu.VMEM((CHUNK, 128), f32),
        pltpu.VMEM((CHUNK, 16), f32),
        pltpu.VMEM((128,), f32),
        pltpu.SemaphoreType.DMA,
        pltpu.SemaphoreType.DMA,
        pltpu.SemaphoreType.DMA,
    ],
)
def _pet_kernel(row_hbm, col_hbm, ewt_hbm, h_hbm, w_hbm,
                pet_out,
                rowb0, colb0, rowb1, colb1, ewtb0, ewtb1, petb,
                hrb0, hcb0, hrb1, hcb1, sb, wb, gsem0, gsem1, isem):
    core, sid, wid = _ids()
    pltpu.sync_copy(w_hbm, wb)
    wv = [wb[pl.ds(k * 16, 16)] for k in range(8)]
    lanes = lax.iota(i32, 16)

    sets = ((rowb0, colb0, ewtb0, hrb0, hcb0, gsem0),
            (rowb1, colb1, ewtb1, hrb1, hcb1, gsem1))

    def _prefetch(c, s):
        rowb, colb, ewtb, hrb, hcb, gsem = s
        blk = wid * RPW + c
        pltpu.async_copy(row_hbm.at[pl.ds(blk, 1)], rowb, isem)
        pltpu.async_copy(col_hbm.at[pl.ds(blk, 1)], colb, isem)
        pltpu.async_copy(ewt_hbm.at[pl.ds(blk, 1)], ewtb, isem)
        pltpu.make_async_copy(row_hbm.at[pl.ds(blk, 1)], rowb, isem).wait()
        pltpu.make_async_copy(col_hbm.at[pl.ds(blk, 1)], colb, isem).wait()
        pltpu.make_async_copy(ewt_hbm.at[pl.ds(blk, 1)], ewtb, isem).wait()
        pltpu.async_copy(h_hbm.at[rowb.at[0]], hrb, gsem)
        pltpu.async_copy(h_hbm.at[colb.at[0]], hcb, gsem)

    def _process(c, s):
        rowb, colb, ewtb, hrb, hcb, gsem = s
        blk = wid * RPW + c
        pltpu.make_async_copy(h_hbm.at[rowb.at[0]], hrb, gsem).wait()
        pltpu.make_async_copy(h_hbm.at[colb.at[0]], hcb, gsem).wait()

        def pbody(j, _):
            s16 = jnp.zeros((16,), f32)
            for k in range(8):
                sl = pl.ds(k * 16, 16)
                s16 = s16 + jnp.abs(hrb[j, sl] - hcb[j, sl]) * wv[k]
            sb[j, :] = s16
            return 0

        lax.fori_loop(0, CHUNK, pbody, 0, unroll=2)
        for g in range(CHUNK // 16):
            eidx = g * 16 + lanes
            tot = jnp.zeros((16,), f32)
            for k in range(16):
                tot = tot + plsc.load_gather(
                    sb, [eidx, jnp.full((16,), k, i32)])
            sl = pl.ds(g * 16, 16)
            petb[0, sl] = tot + ewtb[0, sl]
        pltpu.sync_copy(petb, pet_out.at[pl.ds(blk, 1)])

    _prefetch(0, sets[0])

    def body(k, _):
        c = 2 * k
        _prefetch(c + 1, sets[1])
        _process(c, sets[0])

        @pl.when(k < RPW // 2 - 1)
        def _():
            _prefetch(c + 2, sets[0])

        _process(c + 1, sets[1])
        return 0

    lax.fori_loop(0, RPW // 2, body, 0)


def _tc(body, out_shapes, *ins):
    return pl.pallas_call(body, out_shape=out_shapes)(*ins)


def _tca_body(x_ref, du_ref, w_ref, pv_ref, ew_ref,
              s1_ref, s1m_ref, t1_ref, self1_ref, ewt_ref):
    x = x_ref[...]
    s1 = jnp.dot(x, du_ref[...], preferred_element_type=f32)
    s1_ref[...] = s1
    s1m_ref[...] = s1 - pv_ref[0]
    xw = jnp.dot(x, w_ref[...], preferred_element_type=f32)
    m0 = xw[:, :128]
    m1 = xw[:, 128:]
    t1_ref[...] = jnp.concatenate([m0 - m1, m1], axis=1)
    self1_ref[...] = pv_ref[1] * m0 + pv_ref[2] * m1
    ewt_ref[...] = ew_ref[...] * pv_ref[3] + pv_ref[4]


def _tcb_body(acc_ref, self1_ref, degE_ref, cntE_ref, b_ref, du_ref, w_ref,
              pv_ref,
              dinv_ref, cnt_ref, s2_ref, s2m_ref, t2_ref, self2_ref):
    deg = degE_ref[0] + degE_ref[1] + 1.0
    dinv_ref[...] = jnp.where(deg > 0, lax.rsqrt(deg), 0.0)
    cnt0 = cntE_ref[0] + cntE_ref[1] + 1.0
    cnt_ref[...] = cnt0
    cnt = jnp.maximum(cnt0, 1.0)
    h1 = (acc_ref[0] + acc_ref[1] + self1_ref[...]) / cnt[:, None]
    h1 = jnp.maximum(h1 + b_ref[...][None, :], 0.0)
    s2 = jnp.dot(h1, du_ref[...], preferred_element_type=f32)
    s2_ref[...] = s2
    s2m_ref[...] = s2 - pv_ref[0]
    xw = jnp.dot(h1, w_ref[...], preferred_element_type=f32)
    m0 = xw[:, :64]
    m1 = xw[:, 64:]
    t2_ref[...] = jnp.concatenate([m0 - m1, m1], axis=1)
    self2_ref[...] = pv_ref[1] * m0 + pv_ref[2] * m1


def _tcc_body(acc_ref, self2_ref, cnt_ref, b2_ref, wl_ref, bl_ref, we_ref,
              be_ref, dinv_ref,
              h_ref, hw_ref, selfb_ref):
    cnt = jnp.maximum(cnt_ref[...], 1.0)
    h2 = (acc_ref[0, :, :64] + acc_ref[1, :, :64]
          + self2_ref[...]) / cnt[:, None]
    h2 = jnp.maximum(h2 + b2_ref[...][None, :], 0.0)
    h = jnp.dot(h2, wl_ref[...], preferred_element_type=f32) \
        + bl_ref[...][None, :]
    h_ref[...] = h
    hw = jnp.dot(h, we_ref[...], preferred_element_type=f32)
    hw_ref[...] = hw
    d2 = dinv_ref[...] * dinv_ref[...]
    selfb_ref[...] = d2[:, None] * hw + be_ref[...][None, :]


def _tcg_body(acc_ref, selfb_ref, w_ref, b_ref, dinv_ref,
              hw_ref, selfb2_ref):
    e = jnp.maximum(acc_ref[0] + acc_ref[1] + selfb_ref[...], 0.0)
    hw = jnp.dot(e, w_ref[...], preferred_element_type=f32)
    hw_ref[...] = hw
    d2 = dinv_ref[...] * dinv_ref[...]
    selfb2_ref[...] = d2[:, None] * hw + b_ref[...][None, :]


def _tce_body(acc_ref, selfb_ref, w_ref, b_ref, dinv_ref,
              mu_ref, lv_ref, hw_ref, selfb2_ref):
    e = jnp.maximum(acc_ref[0] + acc_ref[1] + selfb_ref[...], 0.0)
    mu = e[:, :64]
    mu_ref[...] = mu
    lv_ref[...] = e[:, 64:]
    hw = jnp.dot(mu, w_ref[...], preferred_element_type=f32)
    hw_ref[...] = hw
    d2 = dinv_ref[...] * dinv_ref[...]
    selfb2_ref[...] = d2[:, None] * hw + b_ref[...][None, :]


def _tcf_body(acc_ref, selfb_ref, recon_ref):
    recon_ref[...] = jnp.tanh(acc_ref[0] + acc_ref[1] + selfb_ref[...])


def kernel(x, edge_index, edge_weight, params):
    n, df = x.shape
    e = edge_index.shape[1]

    pad_e = EPAD - e
    row = jnp.concatenate(
        [edge_index[0], jnp.full((pad_e,), n, i32)]).reshape(MROWS, CHUNK)
    col = jnp.concatenate(
        [edge_index[1], jnp.full((pad_e,), n, i32)]).reshape(MROWS, CHUNK)
    ew2 = jnp.concatenate(
        [edge_weight[:, 0], jnp.zeros((pad_e,), f32)]).reshape(MROWS, CHUNK)
    cval = jnp.concatenate(
        [jnp.ones((e,), f32), jnp.zeros((pad_e,), f32)]).reshape(MROWS, CHUNK)
    wv2 = jnp.stack([ew2, cval], axis=1).reshape(2 * MROWS, CHUNK)
    xp = jnp.pad(x, ((0, NP - n), (0, 0)))

    def _mat(p, key):
        w = p['w_mu'] + jnp.exp(0.5 * p['w_logvar']) * jax.random.normal(
            key, p['w_mu'].shape, dtype=f32)
        b = p['b_mu'] + jnp.exp(0.5 * p['b_logvar']) * jax.random.normal(
            jax.random.fold_in(key, 1), p['b_mu'].shape, dtype=f32)
        return w, b

    kk = jax.random.key(42)
    we1, be1 = _mat(params['enc1'], jax.random.fold_in(kk, 0))
    we2, be2 = _mat(params['enc2'], jax.random.fold_in(kk, 1))
    wd1, bd1 = _mat(params['dec1'], jax.random.fold_in(kk, 2))
    wd2, bd2 = _mat(params['dec2'], jax.random.fold_in(kk, 3))

    f1, f2 = params['feast1'], params['feast2']
    du1 = f1['u'][:, 0] - f1['u'][:, 1]
    q1 = jax.nn.softmax(f1['c'])
    du2 = f2['u'][:, 0] - f2['u'][:, 1]
    q2 = jax.nn.softmax(f2['c'])
    etpw = params['etp']['W']
    pva = jnp.stack([f1['c'][0] - f1['c'][1], q1[0], q1[1],
                     etpw[df, 0], params['etp']['b'][0]])
    pvb = jnp.stack([f2['c'][0] - f2['c'][1], q2[0], q2[1]])
    wvec = etpw[:df, 0]

    degE, cntE = _deg_kernel(col, wv2)

    s1, s1m, t1, self1, ewt = _tc(
        _tca_body,
        (jax.ShapeDtypeStruct((NP,), f32),
         jax.ShapeDtypeStruct((NP,), f32),
         jax.ShapeDtypeStruct((NP, 256), f32),
         jax.ShapeDtypeStruct((NP, 128), f32),
         jax.ShapeDtypeStruct((MROWS, CHUNK), f32)),
        xp, du1, f1['W'], pva, ew2)

    acc1 = _f1_kernel(row.reshape(F1ROWS, F1C), col.reshape(F1ROWS, F1C),
                      t1, s1, s1m)

    dinv, cnt, s2, s2m, t2, self2 = _tc(
        _tcb_body,
        (jax.ShapeDtypeStruct((NP,), f32),
         jax.ShapeDtypeStruct((NP,), f32),
         jax.ShapeDtypeStruct((NP,), f32),
         jax.ShapeDtypeStruct((NP,), f32),
         jax.ShapeDtypeStruct((NP, 128), f32),
         jax.ShapeDtypeStruct((NP, 64), f32)),
        acc1, self1, degE, cntE, f1['b'], du2, f2['W'], pvb)

    acc2 = _f2_kernel(row.reshape(F2ROWS, F2C), col.reshape(F2ROWS, F2C),
                      t2, s2, s2m)

    h, hw1, selfb1 = _tc(
        _tcc_body,
        (jax.ShapeDtypeStruct((NP, 128), f32),
         jax.ShapeDtypeStruct((NP, 128), f32),
         jax.ShapeDtypeStruct((NP, 128), f32)),
        acc2, self2, cnt, f2['b'], params['linear']['W'],
        params['linear']['b'], we1, be1, dinv)

    accg1, norm = _gcn_norm_kernel(row, col, ew2, dinv, hw1)
    pet2 = _pet_kernel(row, col, ewt, h, wvec)

    hw2, selfb2 = _tc(
        _tcg_body,
        (jax.ShapeDtypeStruct((NP, 128), f32),
         jax.ShapeDtypeStruct((NP, 128), f32)),
        accg1, selfb1, we2, be2, dinv)

    accg2 = _gcn_kernel(row, col, norm, hw2)

    mu, logvar, hw3, selfb3 = _tc(
        _tce_body,
        (jax.ShapeDtypeStruct((NP, 64), f32),
         jax.ShapeDtypeStruct((NP, 64), f32),
         jax.ShapeDtypeStruct((NP, 128), f32),
         jax.ShapeDtypeStruct((NP, 128), f32)),
        accg2, selfb2, wd1, bd1, dinv)

    accg3 = _gcn_kernel(row, col, norm, hw3)

    hw4, selfb4 = _tc(
        _tcg_body,
        (jax.ShapeDtypeStruct((NP, 128), f32),
         jax.ShapeDtypeStruct((NP, 128), f32)),
        accg3, selfb3, wd2, bd2, dinv)

    accg4 = _gcn_kernel(row, col, norm, hw4)

    recon = _tc(
        _tcf_body,
        jax.ShapeDtypeStruct((NP, 128), f32),
        accg4, selfb4)

    pet = pet2.reshape(-1)[:e, None]
    return recon[:n], mu[:n], logvar[:n], pet


# speedup vs baseline: 1.1501x; 1.0772x over previous
import functools

import jax
import jax.numpy as jnp
from jax import lax
from jax.experimental import pallas as pl
from jax.experimental.pallas import tpu as pltpu
from jax.experimental.pallas import tpu_sc as plsc

f32 = jnp.float32
i32 = jnp.int32

N = 10000
NP = 10240
E = 320000
NC = 2
NS = 16
NWORK = NC * NS
CHUNK = 128
MROWS = 2560
EPAD = MROWS * CHUNK
RPW = MROWS // NWORK
ROWS_PER_TILE = NP // NS

F1C = 32
F1ROWS = EPAD // F1C
F1RPW = F1ROWS // NWORK
F2C = 64
F2ROWS = EPAD // F2C
F2RPW = F2ROWS // NWORK

_MESH = plsc.VectorSubcoreMesh(core_axis_name="c", subcore_axis_name="s",
                               num_cores=NC, num_subcores=NS)
_CP = pltpu.CompilerParams(needs_layout_passes=False)


def _ids():
    core = lax.axis_index("c")
    sid = lax.axis_index("s")
    wid = sid * NC + core
    return core, sid, wid


def _zero_vmem(ref, rows, width):
    z = jnp.zeros((16,), f32)

    def body(j, _):
        for k in range(width // 16):
            ref[j, pl.ds(k * 16, 16)] = z
        return 0

    lax.fori_loop(0, rows, body, 0, unroll=4)


def _zero_acc(zsrc, rows, acc_sh, sid):
    for t in range(ROWS_PER_TILE // rows):
        pltpu.sync_copy(
            zsrc, acc_sh.at[pl.ds(sid * ROWS_PER_TILE + t * rows, rows)])


@functools.partial(
    pl.kernel,
    out_type=(
        jax.ShapeDtypeStruct((NC, NP), f32),
        jax.ShapeDtypeStruct((NC, NP), f32),
    ),
    mesh=_MESH,
    compiler_params=_CP,
    scratch_types=[
        pltpu.VMEM((1, CHUNK), i32),
        pltpu.VMEM((1, CHUNK), i32),
        pltpu.VMEM((2, CHUNK), f32),
        pltpu.VMEM((2, CHUNK), f32),
        pltpu.VMEM((ROWS_PER_TILE,), f32),
        pltpu.VMEM_SHARED((NP,), f32),
        pltpu.VMEM_SHARED((NP,), f32),
        pltpu.SemaphoreType.DMA,
        pltpu.SemaphoreType.DMA,
    ],
)
def _deg_kernel(col_hbm, wv_hbm, deg_out, cnt_out,
                colb0, colb1, wvb0, wvb1, zb, deg_sh, cnt_sh, sem0, sem1):
    core, sid, wid = _ids()
    z = jnp.zeros((16,), f32)

    def zbody(j, _):
        zb[pl.ds(j * 16, 16)] = z
        return 0

    lax.fori_loop(0, ROWS_PER_TILE // 16, zbody, 0, unroll=4)
    pltpu.sync_copy(zb, deg_sh.at[pl.ds(sid * ROWS_PER_TILE, ROWS_PER_TILE)])
    pltpu.sync_copy(zb, cnt_sh.at[pl.ds(sid * ROWS_PER_TILE, ROWS_PER_TILE)])
    plsc.subcore_barrier()

    sets = ((colb0, wvb0, sem0), (colb1, wvb1, sem1))

    def _prefetch(c, s):
        colb, wvb, sem = s
        blk = wid * RPW + c
        pltpu.async_copy(col_hbm.at[pl.ds(blk, 1)], colb, sem)
        pltpu.async_copy(wv_hbm.at[pl.ds(2 * blk, 2)], wvb, sem)

    def _drain(s):
        colb, wvb, sem = s
        pltpu.make_async_copy(col_hbm.at[pl.ds(0, 1)], colb, sem).wait()
        pltpu.make_async_copy(wv_hbm.at[pl.ds(0, 2)], wvb, sem).wait()

    def _process(s):
        colb, wvb, _ = s
        pltpu.sync_copy(wvb.at[0], deg_sh.at[colb.at[0]], add=True)
        pltpu.sync_copy(wvb.at[1], cnt_sh.at[colb.at[0]], add=True)

    _prefetch(0, sets[0])

    def body(k, _):
        c = 2 * k
        _prefetch(c + 1, sets[1])
        _drain(sets[0])
        _process(sets[0])

        @pl.when(k < RPW // 2 - 1)
        def _():
            _prefetch(c + 2, sets[0])

        _drain(sets[1])
        _process(sets[1])
        return 0

    lax.fori_loop(0, RPW // 2, body, 0)
    plsc.subcore_barrier()
    sl = pl.ds(sid * ROWS_PER_TILE, ROWS_PER_TILE)
    pltpu.sync_copy(deg_sh.at[sl], deg_out.at[core, sl])
    pltpu.sync_copy(cnt_sh.at[sl], cnt_out.at[core, sl])


F1REAL = E // F1C


@functools.partial(
    pl.kernel,
    out_type=jax.ShapeDtypeStruct((NC, NP, 128), f32),
    mesh=_MESH,
    compiler_params=_CP,
    scratch_types=[
        pltpu.VMEM((1, F1C), i32),
        pltpu.VMEM((1, F1C), i32),
        pltpu.VMEM((1, F1C), i32),
        pltpu.VMEM((1, F1C), i32),
        pltpu.VMEM((1, F1C + 16), f32),
        pltpu.VMEM((F1C, 256), f32),
        pltpu.VMEM((F1C, 256), f32),
        pltpu.VMEM((F1C, 128), f32),
        pltpu.VMEM((NP,), f32),
        pltpu.VMEM((NP,), f32),
        pltpu.VMEM_SHARED((NP, 128), f32),
        pltpu.SemaphoreType.DMA,
        pltpu.SemaphoreType.DMA,
        pltpu.SemaphoreType.DMA,
    ],
)
def _f1_kernel(row_hbm, col_hbm, t1_hbm, s1_hbm, s1m_hbm,
               acc_out,
               rowb0, colb0, rowb1, colb1, qb, gb0, gb1,
               msgb, s1l, s1ml, acc_sh, gsem0, gsem1, isem):
    core, sid, wid = _ids()
    pltpu.sync_copy(s1_hbm, s1l)
    pltpu.sync_copy(s1m_hbm, s1ml)
    _zero_vmem(msgb, F1C, 128)
    _zero_acc(msgb, F1C, acc_sh, sid)
    plsc.subcore_barrier()

    sets = ((rowb0, colb0, gb0, gsem0), (rowb1, colb1, gb1, gsem1))

    def _prefetch(c, s):
        rowb, colb, gb, gsem = s
        blk = wid * F1RPW + c
        pltpu.async_copy(row_hbm.at[pl.ds(blk, 1)], rowb, isem)
        pltpu.async_copy(col_hbm.at[pl.ds(blk, 1)], colb, isem)
        pltpu.make_async_copy(row_hbm.at[pl.ds(blk, 1)], rowb, isem).wait()
        pltpu.make_async_copy(col_hbm.at[pl.ds(blk, 1)], colb, isem).wait()
        pltpu.async_copy(t1_hbm.at[rowb.at[0]], gb, gsem)

    def _process(c, s):
        rowb, colb, gb, gsem = s
        for g in range(F1C // 16):
            sl = pl.ds(g * 16, 16)
            sr = plsc.load_gather(s1l, [rowb[0, sl]])
            sc = plsc.load_gather(s1ml, [colb[0, sl]])
            qb[0, sl] = 1.0 / (1.0 + jnp.exp(sc - sr))
        pltpu.make_async_copy(t1_hbm.at[rowb.at[0]], gb, gsem).wait()

        def mbody(j, _):
            q = jnp.full((16,), qb[0, pl.ds(j, 16)][0], f32)
            for k in range(8):
                sl = pl.ds(k * 16, 16)
                msgb[j, sl] = q * gb[j, sl] + gb[j, pl.ds(128 + k * 16, 16)]
            return 0

        lax.fori_loop(0, F1C, mbody, 0, unroll=2)
        pltpu.sync_copy(msgb, acc_sh.at[colb.at[0]], add=True)

    _prefetch(0, sets[0])

    def body(k, _):
        c = 2 * k
        _prefetch(c + 1, sets[1])
        _process(c, sets[0])

        @pl.when(k < F1RPW // 2 - 1)
        def _():
            _prefetch(c + 2, sets[0])

        _process(c + 1, sets[1])
        return 0

    lax.fori_loop(0, F1RPW // 2, body, 0)
    plsc.subcore_barrier()
    sl = pl.ds(sid * ROWS_PER_TILE, ROWS_PER_TILE)
    pltpu.sync_copy(acc_sh.at[sl], acc_out.at[core, sl])


@functools.partial(
    pl.kernel,
    out_type=jax.ShapeDtypeStruct((NC, NP, 128), f32),
    mesh=_MESH,
    compiler_params=_CP,
    scratch_types=[
        pltpu.VMEM((1, F2C), i32),
        pltpu.VMEM((1, F2C), i32),
        pltpu.VMEM((1, F2C), i32),
        pltpu.VMEM((1, F2C), i32),
        pltpu.VMEM((1, F2C + 16), f32),
        pltpu.VMEM((F2C, 128), f32),
        pltpu.VMEM((F2C, 128), f32),
        pltpu.VMEM((F2C, 128), f32),
        pltpu.VMEM((NP,), f32),
        pltpu.VMEM((NP,), f32),
        pltpu.VMEM_SHARED((NP, 128), f32),
        pltpu.SemaphoreType.DMA,
        pltpu.SemaphoreType.DMA,
        pltpu.SemaphoreType.DMA,
    ],
)
def _f2_kernel(row_hbm, col_hbm, t2_hbm, s2_hbm, s2m_hbm,
               acc_out,
               rowb0, colb0, rowb1, colb1, qb, gb0, gb1, msgb, s2l, s2ml,
               acc_sh, gsem0, gsem1, isem):
    core, sid, wid = _ids()
    pltpu.sync_copy(s2_hbm, s2l)
    pltpu.sync_copy(s2m_hbm, s2ml)
    _zero_vmem(msgb, F2C, 128)
    _zero_acc(msgb, F2C, acc_sh, sid)
    plsc.subcore_barrier()

    sets = ((rowb0, colb0, gb0, gsem0), (rowb1, colb1, gb1, gsem1))

    def _prefetch(c, s):
        rowb, colb, gb, gsem = s
        blk = wid * F2RPW + c
        pltpu.async_copy(row_hbm.at[pl.ds(blk, 1)], rowb, isem)
        pltpu.async_copy(col_hbm.at[pl.ds(blk, 1)], colb, isem)
        pltpu.make_async_copy(row_hbm.at[pl.ds(blk, 1)], rowb, isem).wait()
        pltpu.make_async_copy(col_hbm.at[pl.ds(blk, 1)], colb, isem).wait()
        pltpu.async_copy(t2_hbm.at[rowb.at[0]], gb, gsem)

    def _process(s):
        rowb, colb, gb, gsem = s
        for g in range(F2C // 16):
            sl = pl.ds(g * 16, 16)
            sr = plsc.load_gather(s2l, [rowb[0, sl]])
            sc = plsc.load_gather(s2ml, [colb[0, sl]])
            qb[0, sl] = 1.0 / (1.0 + jnp.exp(sc - sr))
        pltpu.make_async_copy(t2_hbm.at[rowb.at[0]], gb, gsem).wait()

        def mbody(j, _):
            q = jnp.full((16,), qb[0, pl.ds(j, 16)][0], f32)
            for k in range(4):
                sl = pl.ds(k * 16, 16)
                msgb[j, sl] = q * gb[j, sl] + gb[j, pl.ds(64 + k * 16, 16)]
            return 0

        lax.fori_loop(0, F2C, mbody, 0, unroll=2)
        pltpu.sync_copy(msgb, acc_sh.at[colb.at[0]], add=True)

    _prefetch(0, sets[0])

    def body(k, _):
        c = 2 * k
        _prefetch(c + 1, sets[1])
        _process(sets[0])

        @pl.when(k < F2RPW // 2 - 1)
        def _():
            _prefetch(c + 2, sets[0])

        _process(sets[1])
        return 0

    lax.fori_loop(0, F2RPW // 2, body, 0)
    plsc.subcore_barrier()
    sl = pl.ds(sid * ROWS_PER_TILE, ROWS_PER_TILE)
    pltpu.sync_copy(acc_sh.at[sl], acc_out.at[core, sl])


def _gcn_scale_scatter(s, acc_sh, tab_hbm):
    rowb, colb, nb, gb, gsem = s
    pltpu.make_async_copy(tab_hbm.at[rowb.at[0]], gb, gsem).wait()

    def mbody(j, _):
        nv = jnp.full((16,), nb[0, pl.ds(j, 16)][0], f32)
        for k in range(8):
            sl = pl.ds(k * 16, 16)
            gb[j, sl] = nv * gb[j, sl]
        return 0

    lax.fori_loop(0, CHUNK, mbody, 0, unroll=2)
    pltpu.sync_copy(gb, acc_sh.at[colb.at[0]], add=True)


@functools.partial(
    pl.kernel,
    out_type=jax.ShapeDtypeStruct((NC, NP, 128), f32),
    mesh=_MESH,
    compiler_params=_CP,
    scratch_types=[
        pltpu.VMEM((1, CHUNK), i32),
        pltpu.VMEM((1, CHUNK), i32),
        pltpu.VMEM((1, CHUNK), i32),
        pltpu.VMEM((1, CHUNK), i32),
        pltpu.VMEM((1, CHUNK + 16), f32),
        pltpu.VMEM((1, CHUNK + 16), f32),
        pltpu.VMEM((CHUNK, 128), f32),
        pltpu.VMEM((CHUNK, 128), f32),
        pltpu.VMEM_SHARED((NP, 128), f32),
        pltpu.SemaphoreType.DMA,
        pltpu.SemaphoreType.DMA,
        pltpu.SemaphoreType.DMA,
    ],
)
def _gcn_kernel(row_hbm, col_hbm, ew_hbm, tab_hbm,
                acc_out,
                rowb0, colb0, rowb1, colb1, nb0, nb1, gb0, gb1,
                acc_sh, gsem0, gsem1, isem):
    core, sid, wid = _ids()
    _zero_vmem(gb0, CHUNK, 128)
    _zero_acc(gb0, CHUNK, acc_sh, sid)
    plsc.subcore_barrier()

    sets = ((rowb0, colb0, nb0, gb0, gsem0), (rowb1, colb1, nb1, gb1, gsem1))

    def _prefetch(c, s):
        rowb, colb, nb, gb, gsem = s
        blk = wid * RPW + c
        pltpu.async_copy(row_hbm.at[pl.ds(blk, 1)], rowb, isem)
        pltpu.async_copy(col_hbm.at[pl.ds(blk, 1)], colb, isem)
        pltpu.async_copy(ew_hbm.at[pl.ds(blk, 1)],
                         nb.at[:, pl.ds(0, CHUNK)], isem)
        pltpu.make_async_copy(row_hbm.at[pl.ds(blk, 1)], rowb, isem).wait()
        pltpu.make_async_copy(col_hbm.at[pl.ds(blk, 1)], colb, isem).wait()
        pltpu.make_async_copy(ew_hbm.at[pl.ds(blk, 1)],
                              nb.at[:, pl.ds(0, CHUNK)], isem).wait()
        pltpu.async_copy(tab_hbm.at[rowb.at[0]], gb, gsem)

    _prefetch(0, sets[0])

    def body(k, _):
        c = 2 * k
        _prefetch(c + 1, sets[1])
        _gcn_scale_scatter(sets[0], acc_sh, tab_hbm)

        @pl.when(k < RPW // 2 - 1)
        def _():
            _prefetch(c + 2, sets[0])

        _gcn_scale_scatter(sets[1], acc_sh, tab_hbm)
        return 0

    lax.fori_loop(0, RPW // 2, body, 0)
    plsc.subcore_barrier()
    sl = pl.ds(sid * ROWS_PER_TILE, ROWS_PER_TILE)
    pltpu.sync_copy(acc_sh.at[sl], acc_out.at[core, sl])


E1C = 32
E1ROWS = EPAD // E1C
E1RPW = E1ROWS // NWORK


@functools.partial(
    pl.kernel,
    out_type=(
        jax.ShapeDtypeStruct((NC, NP, 128), f32),
        jax.ShapeDtypeStruct((E1ROWS, E1C), f32),
    ),
    mesh=_MESH,
    compiler_params=_CP,
    scratch_types=[
        pltpu.VMEM((1, E1C), i32),
        pltpu.VMEM((1, E1C), i32),
        pltpu.VMEM((1, E1C), i32),
        pltpu.VMEM((1, E1C), i32),
        pltpu.VMEM((1, E1C + 16), f32),
        pltpu.VMEM((1, E1C + 16), f32),
        pltpu.VMEM((1, E1C), f32),
        pltpu.VMEM((1, E1C), f32),
        pltpu.VMEM((E1C, 256), f32),
        pltpu.VMEM((E1C, 256), f32),
        pltpu.VMEM((E1C, 128), f32),
        pltpu.VMEM((E1C, 128), f32),
        pltpu.VMEM((E1C, 128), f32),
        pltpu.VMEM((E1C, 16), f32),
        pltpu.VMEM((1, E1C), f32),
        pltpu.VMEM((128,), f32),
        pltpu.VMEM_SHARED((NP, 128), f32),
        pltpu.SemaphoreType.DMA,
        pltpu.SemaphoreType.DMA,
        pltpu.SemaphoreType.DMA,
    ],
)
def _enc1_pet_kernel(row_hbm, col_hbm, ew_hbm, tab_hbm, h_hbm, w_hbm,
                     acc_out, pet_out,
                     rowb0, colb0, rowb1, colb1, nb0, nb1, ewl0, ewl1,
                     gb0, gb1, hcb0, hcb1, msgb, sb, petb, wb,
                     acc_sh, gsem0, gsem1, isem):
    core, sid, wid = _ids()
    pltpu.sync_copy(w_hbm, wb)
    wv = [wb[pl.ds(k * 16, 16)] for k in range(8)]
    lanes = lax.iota(i32, 16)
    _zero_vmem(msgb, E1C, 128)
    _zero_acc(msgb, E1C, acc_sh, sid)
    plsc.subcore_barrier()

    sets = ((rowb0, colb0, nb0, ewl0, gb0, hcb0, gsem0),
            (rowb1, colb1, nb1, ewl1, gb1, hcb1, gsem1))

    def _prefetch(c, s):
        rowb, colb, nb, ewl, gb, hcb, gsem = s
        blk = wid * E1RPW + c
        pltpu.async_copy(row_hbm.at[pl.ds(blk, 1)], rowb, isem)
        pltpu.async_copy(col_hbm.at[pl.ds(blk, 1)], colb, isem)
        pltpu.async_copy(ew_hbm.at[pl.ds(blk, 1)], ewl, isem)
        pltpu.make_async_copy(row_hbm.at[pl.ds(blk, 1)], rowb, isem).wait()
        pltpu.make_async_copy(col_hbm.at[pl.ds(blk, 1)], colb, isem).wait()
        pltpu.make_async_copy(ew_hbm.at[pl.ds(blk, 1)], ewl, isem).wait()
        for g in range(E1C // 16):
            sl = pl.ds(g * 16, 16)
            nb[0, sl] = ewl[0, sl]
        pltpu.async_copy(tab_hbm.at[rowb.at[0]], gb, gsem)
        pltpu.async_copy(h_hbm.at[colb.at[0]], hcb, gsem)

    def _process(c, s):
        rowb, colb, nb, ewl, gb, hcb, gsem = s
        blk = wid * E1RPW + c
        pltpu.make_async_copy(tab_hbm.at[rowb.at[0]], gb, gsem).wait()
        pltpu.make_async_copy(h_hbm.at[colb.at[0]], hcb, gsem).wait()

        def mbody(j, _):
            nv = jnp.full((16,), nb[0, pl.ds(j, 16)][0], f32)
            s16 = jnp.zeros((16,), f32)
            for k in range(8):
                sl = pl.ds(k * 16, 16)
                msgb[j, sl] = nv * gb[j, sl]
                s16 = s16 + jnp.abs(
                    gb[j, pl.ds(128 + k * 16, 16)] - hcb[j, sl]) * wv[k]
            sb[j, :] = s16
            return 0

        lax.fori_loop(0, E1C, mbody, 0, unroll=2)
        pltpu.sync_copy(msgb, acc_sh.at[colb.at[0]], add=True)
        for g in range(E1C // 16):
            eidx = g * 16 + lanes
            tot = jnp.zeros((16,), f32)
            for k in range(16):
                tot = tot + plsc.load_gather(
                    sb, [eidx, jnp.full((16,), k, i32)])
            petb[0, pl.ds(g * 16, 16)] = tot
        pltpu.sync_copy(petb, pet_out.at[pl.ds(blk, 1)])

    _prefetch(0, sets[0])

    def body(k, _):
        c = 2 * k
        _prefetch(c + 1, sets[1])
        _process(c, sets[0])

        @pl.when(k < E1RPW // 2 - 1)
        def _():
            _prefetch(c + 2, sets[0])

        _process(c + 1, sets[1])
        return 0

    lax.fori_loop(0, E1RPW // 2, body, 0)
    plsc.subcore_barrier()
    sl = pl.ds(sid * ROWS_PER_TILE, ROWS_PER_TILE)
    pltpu.sync_copy(acc_sh.at[sl], acc_out.at[core, sl])


def _tc(body, out_shapes, *ins):
    return pl.pallas_call(body, out_shape=out_shapes)(*ins)


def _tca_body(x_ref, du_ref, w_ref, pv_ref, ew_ref,
              s1_ref, s1m_ref, t1_ref, self1_ref, ewt_ref):
    x = x_ref[...]
    s1 = jnp.dot(x, du_ref[...], preferred_element_type=f32)
    s1_ref[...] = s1
    s1m_ref[...] = s1 - pv_ref[0]
    xw = jnp.dot(x, w_ref[...], preferred_element_type=f32)
    m0 = xw[:, :128]
    m1 = xw[:, 128:]
    t1_ref[...] = jnp.concatenate([m0 - m1, m1], axis=1)
    self1_ref[...] = pv_ref[1] * m0 + pv_ref[2] * m1
    ewt_ref[...] = ew_ref[...] * pv_ref[3] + pv_ref[4]


def _tcb_body(acc_ref, self1_ref, degE_ref, cntE_ref, b_ref, du_ref, w_ref,
              pv_ref,
              dinv_ref, cnt_ref, s2_ref, s2m_ref, t2_ref, self2_ref):
    deg = degE_ref[0] + degE_ref[1] + 1.0
    dinv_ref[...] = jnp.where(deg > 0, lax.rsqrt(deg), 0.0)
    cnt0 = cntE_ref[0] + cntE_ref[1] + 1.0
    cnt_ref[...] = cnt0
    cnt = jnp.maximum(cnt0, 1.0)
    h1 = (acc_ref[0] + acc_ref[1] + self1_ref[...]) / cnt[:, None]
    h1 = jnp.maximum(h1 + b_ref[...][None, :], 0.0)
    s2 = jnp.dot(h1, du_ref[...], preferred_element_type=f32)
    s2_ref[...] = s2
    s2m_ref[...] = s2 - pv_ref[0]
    xw = jnp.dot(h1, w_ref[...], preferred_element_type=f32)
    m0 = xw[:, :64]
    m1 = xw[:, 64:]
    t2_ref[...] = jnp.concatenate([m0 - m1, m1], axis=1)
    self2_ref[...] = pv_ref[1] * m0 + pv_ref[2] * m1


def _tcc_body(acc_ref, self2_ref, cnt_ref, b2_ref, wl_ref, bl_ref, we_ref,
              dinv_ref,
              h_ref, tab_ref):
    cnt = jnp.maximum(cnt_ref[...], 1.0)
    h2 = (acc_ref[0, :, :64] + acc_ref[1, :, :64]
          + self2_ref[...]) / cnt[:, None]
    h2 = jnp.maximum(h2 + b2_ref[...][None, :], 0.0)
    h = jnp.dot(h2, wl_ref[...], preferred_element_type=f32) \
        + bl_ref[...][None, :]
    h_ref[...] = h
    hw = jnp.dot(h, we_ref[...], preferred_element_type=f32)
    tab_ref[...] = jnp.concatenate(
        [dinv_ref[...][:, None] * hw, h], axis=1)


def _tcg_body(acc_ref, hwp_ref, bp_ref, w_ref, dinv_ref,
              hwn_ref):
    dinv = dinv_ref[...]
    e = dinv[:, None] * (acc_ref[0] + acc_ref[1] + hwp_ref[:, :128])
    e = jnp.maximum(e + bp_ref[...][None, :], 0.0)
    hwn_ref[...] = dinv[:, None] * jnp.dot(
        e, w_ref[...], preferred_element_type=f32)


def _tce_body(acc_ref, hwp_ref, bp_ref, w_ref, dinv_ref,
              mu_ref, lv_ref, hwn_ref):
    dinv = dinv_ref[...]
    e = dinv[:, None] * (acc_ref[0] + acc_ref[1] + hwp_ref[...])
    e = jnp.maximum(e + bp_ref[...][None, :], 0.0)
    mu = e[:, :64]
    mu_ref[...] = mu
    lv_ref[...] = e[:, 64:]
    hwn_ref[...] = dinv[:, None] * jnp.dot(
        mu, w_ref[...], preferred_element_type=f32)


def _tcf_body(acc_ref, hwp_ref, bp_ref, dinv_ref, praw_ref, ewt_ref,
              recon_ref, pet_ref):
    dinv = dinv_ref[...]
    e = dinv[:, None] * (acc_ref[0] + acc_ref[1] + hwp_ref[...])
    recon_ref[...] = jnp.tanh(e + bp_ref[...][None, :])
    pet_ref[...] = praw_ref[...] + ewt_ref[...]


def kernel(x, edge_index, edge_weight, params):
    n, df = x.shape
    e = edge_index.shape[1]

    pad_e = EPAD - e
    row = jnp.concatenate(
        [edge_index[0], jnp.full((pad_e,), n, i32)]).reshape(MROWS, CHUNK)
    col = jnp.concatenate(
        [edge_index[1], jnp.full((pad_e,), n, i32)]).reshape(MROWS, CHUNK)
    ew2 = jnp.concatenate(
        [edge_weight[:, 0], jnp.zeros((pad_e,), f32)]).reshape(MROWS, CHUNK)
    cval = jnp.concatenate(
        [jnp.ones((e,), f32), jnp.zeros((pad_e,), f32)]).reshape(MROWS, CHUNK)
    wv2 = jnp.stack([ew2, cval], axis=1).reshape(2 * MROWS, CHUNK)
    xp = jnp.pad(x, ((0, NP - n), (0, 0)))

    def _mat(p, key):
        w = p['w_mu'] + jnp.exp(0.5 * p['w_logvar']) * jax.random.normal(
            key, p['w_mu'].shape, dtype=f32)
        b = p['b_mu'] + jnp.exp(0.5 * p['b_logvar']) * jax.random.normal(
            jax.random.fold_in(key, 1), p['b_mu'].shape, dtype=f32)
        return w, b

    kk = jax.random.key(42)
    we1, be1 = _mat(params['enc1'], jax.random.fold_in(kk, 0))
    we2, be2 = _mat(params['enc2'], jax.random.fold_in(kk, 1))
    wd1, bd1 = _mat(params['dec1'], jax.random.fold_in(kk, 2))
    wd2, bd2 = _mat(params['dec2'], jax.random.fold_in(kk, 3))

    f1, f2 = params['feast1'], params['feast2']
    du1 = f1['u'][:, 0] - f1['u'][:, 1]
    q1 = jax.nn.softmax(f1['c'])
    du2 = f2['u'][:, 0] - f2['u'][:, 1]
    q2 = jax.nn.softmax(f2['c'])
    etpw = params['etp']['W']
    pva = jnp.stack([f1['c'][0] - f1['c'][1], q1[0], q1[1],
                     etpw[df, 0], params['etp']['b'][0]])
    pvb = jnp.stack([f2['c'][0] - f2['c'][1], q2[0], q2[1]])
    wvec = etpw[:df, 0]

    degE, cntE = _deg_kernel(col, wv2)

    s1, s1m, t1, self1, ewt = _tc(
        _tca_body,
        (jax.ShapeDtypeStruct((NP,), f32),
         jax.ShapeDtypeStruct((NP,), f32),
         jax.ShapeDtypeStruct((NP, 256), f32),
         jax.ShapeDtypeStruct((NP, 128), f32),
         jax.ShapeDtypeStruct((MROWS, CHUNK), f32)),
        xp, du1, f1['W'], pva, ew2)

    acc1 = _f1_kernel(row.reshape(F1ROWS, F1C), col.reshape(F1ROWS, F1C),
                      t1, s1, s1m)

    dinv, cnt, s2, s2m, t2, self2 = _tc(
        _tcb_body,
        (jax.ShapeDtypeStruct((NP,), f32),
         jax.ShapeDtypeStruct((NP,), f32),
         jax.ShapeDtypeStruct((NP,), f32),
         jax.ShapeDtypeStruct((NP,), f32),
         jax.ShapeDtypeStruct((NP, 128), f32),
         jax.ShapeDtypeStruct((NP, 64), f32)),
        acc1, self1, degE, cntE, f1['b'], du2, f2['W'], pvb)

    acc2 = _f2_kernel(row.reshape(F2ROWS, F2C), col.reshape(F2ROWS, F2C),
                      t2, s2, s2m)

    h, tab1 = _tc(
        _tcc_body,
        (jax.ShapeDtypeStruct((NP, 128), f32),
         jax.ShapeDtypeStruct((NP, 256), f32)),
        acc2, self2, cnt, f2['b'], params['linear']['W'],
        params['linear']['b'], we1, dinv)

    accg1, petraw = _enc1_pet_kernel(
        row.reshape(E1ROWS, E1C), col.reshape(E1ROWS, E1C),
        ew2.reshape(E1ROWS, E1C), tab1, h, wvec)

    hw2 = _tc(
        _tcg_body,
        jax.ShapeDtypeStruct((NP, 128), f32),
        accg1, tab1, be1, we2, dinv)

    accg2 = _gcn_kernel(row, col, ew2, hw2)

    mu, logvar, hw3 = _tc(
        _tce_body,
        (jax.ShapeDtypeStruct((NP, 64), f32),
         jax.ShapeDtypeStruct((NP, 64), f32),
         jax.ShapeDtypeStruct((NP, 128), f32)),
        accg2, hw2, be2, wd1, dinv)

    accg3 = _gcn_kernel(row, col, ew2, hw3)

    hw4 = _tc(
        _tcg_body,
        jax.ShapeDtypeStruct((NP, 128), f32),
        accg3, hw3, bd1, wd2, dinv)

    accg4 = _gcn_kernel(row, col, ew2, hw4)

    recon, pet2 = _tc(
        _tcf_body,
        (jax.ShapeDtypeStruct((NP, 128), f32),
         jax.ShapeDtypeStruct((MROWS, CHUNK), f32)),
        accg4, hw4, bd2, dinv, petraw.reshape(MROWS, CHUNK), ewt)

    pet = pet2.reshape(-1)[:e, None]
    return recon[:n], mu[:n], logvar[:n], pet
